# baseline (speedup 1.0000x reference)
import functools

import jax
import jax.numpy as jnp
from jax import lax
from jax.experimental import pallas as pl
from jax.experimental.pallas import tpu as pltpu
from jax.experimental.pallas import tpu_sc as plsc

_T, _H, _W, _D = 32, 24, 24, 768
_L = 16
_NC = 2
_NS = 16


def _sc_body(t_hbm, h_hbm, w_hbm, out_hbm, t_v, h_v, w_v, buf_v, sem0, sem1):
    wid = lax.axis_index("s") * _NC + lax.axis_index("c")
    pltpu.sync_copy(t_hbm.at[pl.ds(wid, 1)], t_v)
    pltpu.sync_copy(h_hbm.at[pl.ds(0, _H)], h_v)
    pltpu.sync_copy(w_hbm.at[pl.ds(0, _W)], w_v)

    sems = (sem0, sem1)

    def compute_block(h, b):
        def c_body(c, carry):
            off = c * _L
            th = t_v[0, pl.ds(off, _L)] + h_v[h, pl.ds(off, _L)]
            for w in range(_W):
                buf_v[b, w, pl.ds(off, _L)] = th + w_v[w, pl.ds(off, _L)]
            return carry

        lax.fori_loop(0, _D // _L, c_body, 0, unroll=False)

    copies = [None, None]
    for h in range(_H):
        b = h % 2
        if copies[b] is not None:
            copies[b].wait()
        compute_block(h, b)
        copies[b] = pltpu.async_copy(buf_v.at[b], out_hbm.at[0, wid, h], sems[b])
    copies[0].wait()
    copies[1].wait()


@functools.cache
def _sc_embed():
    return pl.kernel(
        _sc_body,
        out_type=jax.ShapeDtypeStruct((1, _T, _H, _W, _D), jnp.float32),
        mesh=plsc.VectorSubcoreMesh(
            core_axis_name="c", subcore_axis_name="s", num_cores=_NC, num_subcores=_NS
        ),
        scratch_types=[
            pltpu.VMEM((1, _D), jnp.float32),
            pltpu.VMEM((_H, _D), jnp.float32),
            pltpu.VMEM((_W, _D), jnp.float32),
            pltpu.VMEM((2, _W, _D), jnp.float32),
            pltpu.SemaphoreType.DMA,
            pltpu.SemaphoreType.DMA,
        ],
    )


def kernel(B, T, H, W, t_table, h_table, w_table):
    return _sc_embed()(t_table, h_table, w_table)
